# Initial kernel scaffold; baseline (speedup 1.0000x reference)
import jax
import jax.numpy as jnp
from jax.experimental import pallas as pl


def kernel(node_embedding, edge_embedding, i, j, index_i, index_j, index_k, index_ji, index_kj, lin_c2_w, lin_c2_b, lin_c3_w, lin_c3_b, ln_c2_g, ln_c2_b, ln_c3_g, ln_c3_b, ln_c2_2_g, ln_c2_2_b, ln_c3_2_g, ln_c3_2_b):
    raise NotImplementedError("write your pallas kernel here")


# speedup vs baseline: 1.5473x; 1.5473x over previous
import jax
import jax.numpy as jnp
from jax import lax
from jax.experimental import pallas as pl
from jax.experimental.pallas import tpu as pltpu
from jax.experimental.pallas import tpu_sc as plsc

N = 10000
E = 160000
T = 320000
D = 128

NC = 2
NS = 16
NW = NC * NS

TP = 327680
TW = TP // NW
EW = E // NW
CHT = 80
CHE = 40
NCH_E = EW // CHE
NCH_T = TW // CHT

_MESH = plsc.VectorSubcoreMesh(core_axis_name="c", subcore_axis_name="s")


def _gather_body(node_h, edge_h, i0_h, i1_h, i2_h, i3_h, i4_h, ii_h, ij_h,
                 x3_h, gi_h, gj_h,
                 v0, v1, v2, v3, v4, vi, vj, r0, r1, r2, r3, r4, e0, e1, sem):
    c = lax.axis_index("c")
    s = lax.axis_index("s")
    wid = s * NC + c
    tbase = wid * TW
    ebase = wid * EW
    idx_h = [i0_h, i1_h, i2_h, i3_h, i4_h]
    idx_v = [v0, v1, v2, v3, v4]
    for b in range(5):
        pltpu.sync_copy(idx_h[b].at[pl.ds(tbase, TW)], idx_v[b])
    pltpu.sync_copy(ii_h.at[pl.ds(ebase, EW)], vi)
    pltpu.sync_copy(ij_h.at[pl.ds(ebase, EW)], vj)
    rows = [r0, r1, r2, r3, r4]

    def tri_fire(ci):
        toff = ci * CHT
        cps = []
        for b in range(5):
            tab = node_h if b < 3 else edge_h
            cps.append(pltpu.async_copy(
                tab.at[idx_v[b].at[pl.ds(toff, CHT)]], rows[b], sem))
        return cps, toff

    def tri_store(toff):
        for b in range(5):
            pltpu.sync_copy(rows[b], x3_h.at[b, pl.ds(tbase + toff, CHT)])

    def chunk(ci, carry):
        cps, toff = tri_fire(ci)
        eoff = ci * CHE
        cps.append(pltpu.async_copy(
            node_h.at[vi.at[pl.ds(eoff, CHE)]], e0, sem))
        cps.append(pltpu.async_copy(
            node_h.at[vj.at[pl.ds(eoff, CHE)]], e1, sem))
        for cp in cps:
            cp.wait()
        tri_store(toff)
        pltpu.sync_copy(e0, gi_h.at[pl.ds(ebase + eoff, CHE)])
        pltpu.sync_copy(e1, gj_h.at[pl.ds(ebase + eoff, CHE)])
        return carry

    def chunk_t(ci, carry):
        cps, toff = tri_fire(ci)
        for cp in cps:
            cp.wait()
        tri_store(toff)
        return carry

    lax.fori_loop(0, NCH_E, chunk, 0)
    lax.fori_loop(NCH_E, NCH_T, chunk_t, 0)


def _gather_sc(node_embedding, edge_embedding, idxs, iis, ijs):
    f32 = jnp.float32
    return pl.kernel(
        _gather_body,
        out_type=(
            jax.ShapeDtypeStruct((5, TP, D), f32),
            jax.ShapeDtypeStruct((E, D), f32),
            jax.ShapeDtypeStruct((E, D), f32),
        ),
        mesh=_MESH,
        scratch_types=[
            pltpu.VMEM((TW,), jnp.int32),
            pltpu.VMEM((TW,), jnp.int32),
            pltpu.VMEM((TW,), jnp.int32),
            pltpu.VMEM((TW,), jnp.int32),
            pltpu.VMEM((TW,), jnp.int32),
            pltpu.VMEM((EW,), jnp.int32),
            pltpu.VMEM((EW,), jnp.int32),
            pltpu.VMEM((CHT, D), f32),
            pltpu.VMEM((CHT, D), f32),
            pltpu.VMEM((CHT, D), f32),
            pltpu.VMEM((CHT, D), f32),
            pltpu.VMEM((CHT, D), f32),
            pltpu.VMEM((CHE, D), f32),
            pltpu.VMEM((CHE, D), f32),
            pltpu.SemaphoreType.DMA,
        ],
    )(node_embedding, edge_embedding, *idxs, iis, ijs)


BT = 512


def _ln(x, g, b, eps=1e-5):
    mu = jnp.mean(x, axis=-1, keepdims=True)
    var = jnp.mean((x - mu) ** 2, axis=-1, keepdims=True)
    return (x - mu) * lax.rsqrt(var + eps) * g + b


def _c3_body(x_ref, w_ref, b_ref, g_ref, beta_ref, o_ref):
    acc = jnp.zeros((2 * D, BT), jnp.float32) + b_ref[...]
    for b in range(5):
        acc += lax.dot_general(w_ref[b], x_ref[b], (((1,), (1,)), ((), ())),
                               preferred_element_type=jnp.float32)
    mu = jnp.mean(acc, axis=0, keepdims=True)
    var = jnp.mean((acc - mu) ** 2, axis=0, keepdims=True)
    y = (acc - mu) * lax.rsqrt(var + 1e-5) * g_ref[...] + beta_ref[...]
    o_ref[...] = jax.nn.sigmoid(y[:D]) * jnp.tanh(y[D:])


def _c3_mlp_tc(x3, w3c, b3, g3, beta3):
    return pl.pallas_call(
        _c3_body,
        grid=(TP // BT,),
        in_specs=[
            pl.BlockSpec((5, BT, D), lambda i: (0, i, 0)),
            pl.BlockSpec((5, 2 * D, D), lambda i: (0, 0, 0)),
            pl.BlockSpec((2 * D, 1), lambda i: (0, 0)),
            pl.BlockSpec((2 * D, 1), lambda i: (0, 0)),
            pl.BlockSpec((2 * D, 1), lambda i: (0, 0)),
        ],
        out_specs=pl.BlockSpec((D, BT), lambda i: (0, i)),
        out_shape=jax.ShapeDtypeStruct((D, TP), jnp.float32),
    )(x3, w3c, b3, g3, beta3)


NCHU = TP // 128
CPW = NCHU // NS
NPASS = D // (8 * NC)
EWR = E // NS


def _scatter_body(gt_h, idx2_h, zero_h, s1_h,
                  idx2_v, gv0, zb, wb,
                  a0, a1, a2, a3, a4, a5, a6, a7, sem):
    c = lax.axis_index("c")
    s = lax.axis_index("s")
    accs = [a0, a1, a2, a3, a4, a5, a6, a7]
    pltpu.sync_copy(idx2_h.at[pl.ds(s * CPW, CPW)], idx2_v)
    pltpu.sync_copy(zero_h, zb)

    for p in range(NPASS):
        k = p * NC + c
        f0 = pl.multiple_of(k * 8, 8)
        for f in range(8):
            pltpu.sync_copy(zb.at[pl.ds(0, EWR)],
                            accs[f].at[pl.ds(s * EWR, EWR)])
        plsc.subcore_barrier()

        def chunk(j, carry):
            ch = s * CPW + j
            pltpu.async_copy(
                gt_h.at[pl.ds(f0, 8), pl.ds(ch * 128, 128)], gv0,
                sem).wait()
            for f in range(8):
                pltpu.sync_copy(gv0.at[f], accs[f].at[idx2_v.at[j]],
                                add=True)
            return carry
        lax.fori_loop(0, CPW, chunk, 0)
        plsc.subcore_barrier()

        for f in range(8):
            pltpu.sync_copy(accs[f].at[pl.ds(s * EWR, EWR)], wb)
            pltpu.sync_copy(wb, s1_h.at[pl.ds((k * 8 + f) * E + s * EWR,
                                              EWR)])
        plsc.subcore_barrier()


def _scatter_sc(gt, idx2):
    f32 = jnp.float32
    zero = jnp.zeros((EWR,), f32)
    acc_t = pltpu.VMEM_SHARED((E + 128,), f32)
    return pl.kernel(
        _scatter_body,
        out_type=jax.ShapeDtypeStruct((D * E,), f32),
        mesh=_MESH,
        scratch_types=[
            pltpu.VMEM((CPW, 128), jnp.int32),
            pltpu.VMEM((8, 128), f32),
            pltpu.VMEM((EWR,), f32),
            pltpu.VMEM((EWR,), f32),
            acc_t, acc_t, acc_t, acc_t, acc_t, acc_t, acc_t, acc_t,
            pltpu.SemaphoreType.DMA,
        ],
    )(gt, idx2, zero)


BE = 640


def _final_body(gi_ref, gj_ref, s_ref, edge_ref, w2_ref, b2_ref, g2_ref,
                beta2_ref, g22_ref, beta22_ref, g32_ref, beta32_ref, o_ref):
    prod = gi_ref[...] * gj_ref[...]
    y = lax.dot_general(prod, w2_ref[...], (((1,), (0,)), ((), ())),
                        preferred_element_type=jnp.float32) + b2_ref[...]
    y = _ln(y, g2_ref[...], beta2_ref[...])
    c2e = _ln(jax.nn.sigmoid(y[:, :D]) * jnp.tanh(y[:, D:]),
              g22_ref[...], beta22_ref[...])
    c3e = _ln(s_ref[...], g32_ref[...], beta32_ref[...])
    o_ref[...] = jnp.tanh(edge_ref[...] + c2e + c3e)


def _final_tc(gi, gj, ssum, edge, w2r, b2, g2, beta2, g22, beta22, g32,
              beta32):
    vspec = lambda n: pl.BlockSpec((n,), lambda i: (0,))
    bspec = pl.BlockSpec((BE, D), lambda i: (i, 0))
    return pl.pallas_call(
        _final_body,
        grid=(E // BE,),
        in_specs=[
            bspec, bspec, bspec, bspec,
            pl.BlockSpec((D, 2 * D), lambda i: (0, 0)),
            vspec(2 * D), vspec(2 * D), vspec(2 * D),
            vspec(D), vspec(D), vspec(D), vspec(D),
        ],
        out_specs=bspec,
        out_shape=jax.ShapeDtypeStruct((E, D), jnp.float32),
    )(gi, gj, ssum, edge, w2r, b2, g2, beta2, g22, beta22, g32, beta32)


@jax.jit
def kernel(node_embedding, edge_embedding, i, j, index_i, index_j, index_k,
           index_ji, index_kj, lin_c2_w, lin_c2_b, lin_c3_w, lin_c3_b,
           ln_c2_g, ln_c2_b, ln_c3_g, ln_c3_b,
           ln_c2_2_g, ln_c2_2_b, ln_c3_2_g, ln_c3_2_b):
    pad = jnp.zeros((TP - T,), jnp.int32)
    idxs = [jnp.concatenate([x.astype(jnp.int32), pad])
            for x in (index_i, index_j, index_k, index_ji, index_kj)]
    x3, gi, gj = _gather_sc(node_embedding, edge_embedding, idxs,
                            i.astype(jnp.int32), j.astype(jnp.int32))

    w3c = lin_c3_w.reshape(2 * D, 5, D).transpose(1, 0, 2)
    gt = _c3_mlp_tc(x3, w3c, lin_c3_b.reshape(2 * D, 1),
                    ln_c3_g.reshape(2 * D, 1), ln_c3_b.reshape(2 * D, 1))

    idx2 = jnp.concatenate(
        [index_ji.astype(jnp.int32),
         jnp.full((TP - T,), E, jnp.int32)]).reshape(NCHU, 128)
    s1 = _scatter_sc(gt, idx2)
    ssum = s1.reshape(D, E).T

    return _final_tc(gi, gj, ssum, edge_embedding, lin_c2_w.T, lin_c2_b,
                     ln_c2_g, ln_c2_b, ln_c2_2_g, ln_c2_2_b,
                     ln_c3_2_g, ln_c3_2_b)


# speedup vs baseline: 1.8793x; 1.2146x over previous
import jax
import jax.numpy as jnp
from jax import lax
from jax.experimental import pallas as pl
from jax.experimental.pallas import tpu as pltpu
from jax.experimental.pallas import tpu_sc as plsc

N = 10000
E = 160000
T = 320000
D = 128

NC = 2
NS = 16
NW = NC * NS

TP = 327680
TW = TP // NW
EW = E // NW
CHT = 80
CHE = 40
NCH_E = EW // CHE
NCH_T = TW // CHT

_MESH = plsc.VectorSubcoreMesh(core_axis_name="c", subcore_axis_name="s")


def _gather_body(node_h, edge_h, i0_h, i1_h, i2_h, i3_h, i4_h, ii_h, ij_h,
                 x3_h, gi_h, gj_h,
                 v0, v1, v2, v3, v4, vi, vj, r0, r1, r2, r3, r4, e0, e1,
                 sem, semt, seme):
    c = lax.axis_index("c")
    s = lax.axis_index("s")
    wid = s * NC + c
    tbase = wid * TW
    ebase = wid * EW
    idx_h = [i0_h, i1_h, i2_h, i3_h, i4_h]
    idx_v = [v0, v1, v2, v3, v4]
    for b in range(5):
        pltpu.sync_copy(idx_h[b].at[pl.ds(tbase, TW)], idx_v[b])
    pltpu.sync_copy(ii_h.at[pl.ds(ebase, EW)], vi)
    pltpu.sync_copy(ij_h.at[pl.ds(ebase, EW)], vj)
    rows = [r0, r1, r2, r3, r4]

    def tri_fire(ci):
        toff = ci * CHT
        cps = []
        for b in range(5):
            tab = node_h if b < 3 else edge_h
            cps.append(pltpu.async_copy(
                tab.at[idx_v[b].at[pl.ds(toff, CHT)]], rows[b], sem))
        return cps, toff

    def tri_store(toff):
        for b in range(5):
            pltpu.async_copy(rows[b], x3_h.at[b, pl.ds(tbase + toff, CHT)],
                             semt)

    def drain_tri():
        for b in range(5):
            pltpu.make_async_copy(rows[b], x3_h.at[b, pl.ds(tbase, CHT)],
                                  semt).wait()

    def e_fire(ci):
        eoff = ci * CHE
        return [pltpu.async_copy(node_h.at[vi.at[pl.ds(eoff, CHE)]], e0,
                                 sem),
                pltpu.async_copy(node_h.at[vj.at[pl.ds(eoff, CHE)]], e1,
                                 sem)], eoff

    def e_store(eoff):
        pltpu.async_copy(e0, gi_h.at[pl.ds(ebase + eoff, CHE)], seme)
        pltpu.async_copy(e1, gj_h.at[pl.ds(ebase + eoff, CHE)], seme)

    def drain_e():
        pltpu.make_async_copy(e0, gi_h.at[pl.ds(ebase, CHE)], seme).wait()
        pltpu.make_async_copy(e1, gj_h.at[pl.ds(ebase, CHE)], seme).wait()

    def do_chunk(ci):
        cps, toff = tri_fire(ci)
        cpe, eoff = e_fire(ci)
        for cp in cps + cpe:
            cp.wait()
        tri_store(toff)
        e_store(eoff)

    def do_chunk_t(ci):
        cps, toff = tri_fire(ci)
        for cp in cps:
            cp.wait()
        tri_store(toff)

    do_chunk(0)

    def chunk(ci, carry):
        drain_tri()
        drain_e()
        do_chunk(ci)
        return carry
    lax.fori_loop(1, NCH_E, chunk, 0)
    drain_tri()
    drain_e()
    do_chunk_t(NCH_E)

    def chunk_t(ci, carry):
        drain_tri()
        do_chunk_t(ci)
        return carry
    lax.fori_loop(NCH_E + 1, NCH_T, chunk_t, 0)
    drain_tri()


def _gather_sc(node_embedding, edge_embedding, idxs, iis, ijs):
    f32 = jnp.float32
    return pl.kernel(
        _gather_body,
        out_type=(
            jax.ShapeDtypeStruct((5, TP, D), f32),
            jax.ShapeDtypeStruct((E, D), f32),
            jax.ShapeDtypeStruct((E, D), f32),
        ),
        mesh=_MESH,
        scratch_types=[
            pltpu.VMEM((TW,), jnp.int32),
            pltpu.VMEM((TW,), jnp.int32),
            pltpu.VMEM((TW,), jnp.int32),
            pltpu.VMEM((TW,), jnp.int32),
            pltpu.VMEM((TW,), jnp.int32),
            pltpu.VMEM((EW,), jnp.int32),
            pltpu.VMEM((EW,), jnp.int32),
            pltpu.VMEM((CHT, D), f32),
            pltpu.VMEM((CHT, D), f32),
            pltpu.VMEM((CHT, D), f32),
            pltpu.VMEM((CHT, D), f32),
            pltpu.VMEM((CHT, D), f32),
            pltpu.VMEM((CHE, D), f32),
            pltpu.VMEM((CHE, D), f32),
            pltpu.SemaphoreType.DMA,
            pltpu.SemaphoreType.DMA,
            pltpu.SemaphoreType.DMA,
        ],
    )(node_embedding, edge_embedding, *idxs, iis, ijs)


BT = 512


def _ln(x, g, b, eps=1e-5):
    mu = jnp.mean(x, axis=-1, keepdims=True)
    var = jnp.mean((x - mu) ** 2, axis=-1, keepdims=True)
    return (x - mu) * lax.rsqrt(var + eps) * g + b


def _c3_body(x_ref, w_ref, b_ref, g_ref, beta_ref, o_ref):
    acc = jnp.zeros((2 * D, BT), jnp.float32) + b_ref[...]
    for b in range(5):
        acc += lax.dot_general(w_ref[b], x_ref[b], (((1,), (1,)), ((), ())),
                               preferred_element_type=jnp.float32)
    mu = jnp.mean(acc, axis=0, keepdims=True)
    var = jnp.mean((acc - mu) ** 2, axis=0, keepdims=True)
    y = (acc - mu) * lax.rsqrt(var + 1e-5) * g_ref[...] + beta_ref[...]
    o_ref[...] = jax.nn.sigmoid(y[:D]) * jnp.tanh(y[D:])


def _c3_mlp_tc(x3, w3c, b3, g3, beta3):
    return pl.pallas_call(
        _c3_body,
        grid=(TP // BT,),
        in_specs=[
            pl.BlockSpec((5, BT, D), lambda i: (0, i, 0)),
            pl.BlockSpec((5, 2 * D, D), lambda i: (0, 0, 0)),
            pl.BlockSpec((2 * D, 1), lambda i: (0, 0)),
            pl.BlockSpec((2 * D, 1), lambda i: (0, 0)),
            pl.BlockSpec((2 * D, 1), lambda i: (0, 0)),
        ],
        out_specs=pl.BlockSpec((D, BT), lambda i: (0, i)),
        out_shape=jax.ShapeDtypeStruct((D, TP), jnp.float32),
    )(x3, w3c, b3, g3, beta3)


NCHU = TP // 128
CPW = NCHU // NS
NPASS = D // (8 * NC)
EWR = E // NS


def _scatter_body(gt_h, idx2_h, zero_h, s1_h,
                  idx2_v, gva, gvb, zb, wb, dr,
                  a0, a1, a2, a3, a4, a5, a6, a7, seml, sema, semb):
    c = lax.axis_index("c")
    s = lax.axis_index("s")
    accs = [a0, a1, a2, a3, a4, a5, a6, a7]
    gv = [gva, gvb]
    sems = [sema, semb]
    pltpu.sync_copy(idx2_h.at[pl.ds(s * CPW, CPW)], idx2_v)
    pltpu.sync_copy(zero_h, zb)

    for p in range(NPASS):
        k = p * NC + c
        f0 = pl.multiple_of(k * 8, 8)
        for f in range(8):
            pltpu.sync_copy(zb.at[pl.ds(0, EWR)],
                            accs[f].at[pl.ds(s * EWR, EWR)])
        plsc.subcore_barrier()

        def load(j, buf):
            ch = s * CPW + j
            return pltpu.async_copy(
                gt_h.at[pl.ds(f0, 8), pl.ds(ch * 128, 128)], buf, seml)

        def fire_adds(j, b):
            for f in range(8):
                pltpu.async_copy(gv[b].at[f], accs[f].at[idx2_v.at[j]],
                                 sems[b], add=True)

        def drain_adds(b):
            for f in range(8):
                pltpu.make_async_copy(zero_h.at[pl.ds(0, 128)], dr,
                                      sems[b]).wait()

        for b in range(2):
            load(b, gv[b]).wait()
            fire_adds(b, b)

        def pair(g, carry):
            for b in range(2):
                j = g * 2 + b
                drain_adds(b)
                load(j, gv[b]).wait()
                fire_adds(j, b)
            return carry
        lax.fori_loop(1, CPW // 2, pair, 0)
        for b in range(2):
            drain_adds(b)
        plsc.subcore_barrier()

        for f in range(8):
            pltpu.sync_copy(accs[f].at[pl.ds(s * EWR, EWR)], wb)
            pltpu.sync_copy(wb, s1_h.at[pl.ds((k * 8 + f) * E + s * EWR,
                                              EWR)])
        plsc.subcore_barrier()


def _scatter_sc(gt, idx2):
    f32 = jnp.float32
    zero = jnp.zeros((EWR,), f32)
    acc_t = pltpu.VMEM_SHARED((E + 128,), f32)
    return pl.kernel(
        _scatter_body,
        out_type=jax.ShapeDtypeStruct((D * E,), f32),
        mesh=_MESH,
        scratch_types=[
            pltpu.VMEM((CPW, 128), jnp.int32),
            pltpu.VMEM((8, 128), f32),
            pltpu.VMEM((8, 128), f32),
            pltpu.VMEM((EWR,), f32),
            pltpu.VMEM((EWR,), f32),
            pltpu.VMEM((128,), f32),
            acc_t, acc_t, acc_t, acc_t, acc_t, acc_t, acc_t, acc_t,
            pltpu.SemaphoreType.DMA,
            pltpu.SemaphoreType.DMA,
            pltpu.SemaphoreType.DMA,
        ],
    )(gt, idx2, zero)


BE = 640


def _final_body(gi_ref, gj_ref, s_ref, edge_ref, w2_ref, b2_ref, g2_ref,
                beta2_ref, g22_ref, beta22_ref, g32_ref, beta32_ref, o_ref):
    prod = gi_ref[...] * gj_ref[...]
    y = lax.dot_general(prod, w2_ref[...], (((1,), (0,)), ((), ())),
                        preferred_element_type=jnp.float32) + b2_ref[...]
    y = _ln(y, g2_ref[...], beta2_ref[...])
    c2e = _ln(jax.nn.sigmoid(y[:, :D]) * jnp.tanh(y[:, D:]),
              g22_ref[...], beta22_ref[...])
    c3e = _ln(s_ref[...], g32_ref[...], beta32_ref[...])
    o_ref[...] = jnp.tanh(edge_ref[...] + c2e + c3e)


def _final_tc(gi, gj, ssum, edge, w2r, b2, g2, beta2, g22, beta22, g32,
              beta32):
    vspec = lambda n: pl.BlockSpec((n,), lambda i: (0,))
    bspec = pl.BlockSpec((BE, D), lambda i: (i, 0))
    return pl.pallas_call(
        _final_body,
        grid=(E // BE,),
        in_specs=[
            bspec, bspec, bspec, bspec,
            pl.BlockSpec((D, 2 * D), lambda i: (0, 0)),
            vspec(2 * D), vspec(2 * D), vspec(2 * D),
            vspec(D), vspec(D), vspec(D), vspec(D),
        ],
        out_specs=bspec,
        out_shape=jax.ShapeDtypeStruct((E, D), jnp.float32),
    )(gi, gj, ssum, edge, w2r, b2, g2, beta2, g22, beta22, g32, beta32)


@jax.jit
def kernel(node_embedding, edge_embedding, i, j, index_i, index_j, index_k,
           index_ji, index_kj, lin_c2_w, lin_c2_b, lin_c3_w, lin_c3_b,
           ln_c2_g, ln_c2_b, ln_c3_g, ln_c3_b,
           ln_c2_2_g, ln_c2_2_b, ln_c3_2_g, ln_c3_2_b):
    pad = jnp.zeros((TP - T,), jnp.int32)
    idxs = [jnp.concatenate([x.astype(jnp.int32), pad])
            for x in (index_i, index_j, index_k, index_ji, index_kj)]
    x3, gi, gj = _gather_sc(node_embedding, edge_embedding, idxs,
                            i.astype(jnp.int32), j.astype(jnp.int32))

    w3c = lin_c3_w.reshape(2 * D, 5, D).transpose(1, 0, 2)
    gt = _c3_mlp_tc(x3, w3c, lin_c3_b.reshape(2 * D, 1),
                    ln_c3_g.reshape(2 * D, 1), ln_c3_b.reshape(2 * D, 1))

    idx2 = jnp.concatenate(
        [index_ji.astype(jnp.int32),
         jnp.full((TP - T,), E, jnp.int32)]).reshape(NCHU, 128)
    s1 = _scatter_sc(gt, idx2)
    ssum = s1.reshape(D, E).T

    return _final_tc(gi, gj, ssum, edge_embedding, lin_c2_w.T, lin_c2_b,
                     ln_c2_g, ln_c2_b, ln_c2_2_g, ln_c2_2_b,
                     ln_c3_2_g, ln_c3_2_b)


# speedup vs baseline: 1.8818x; 1.0013x over previous
import jax
import jax.numpy as jnp
from jax import lax
from jax.experimental import pallas as pl
from jax.experimental.pallas import tpu as pltpu
from jax.experimental.pallas import tpu_sc as plsc

N = 10000
E = 160000
T = 320000
D = 128

NC = 2
NS = 16
NW = NC * NS

TP = 327680
TW = TP // NW
EW = E // NW
CHT = 80
CHE = 40
NCH_E = EW // CHE
NCH_T = TW // CHT

_MESH = plsc.VectorSubcoreMesh(core_axis_name="c", subcore_axis_name="s")


def _gather_body(node_h, edge_h, i0_h, i1_h, i2_h, i3_h, i4_h,
                 ii_h, ij_h, x3_h, gi_h, gj_h,
                 v0, v1, v2, v3, v4, vi, vj, r0, r1, r2, r3, r4, e0, e1,
                 sem, semt, seme):
    c = lax.axis_index("c")
    s = lax.axis_index("s")
    wid = s * NC + c
    tbase = wid * TW
    ebase = wid * EW
    idx_h = [i0_h, i1_h, i2_h, i3_h, i4_h]
    idx_v = [v0, v1, v2, v3, v4]
    for b in range(5):
        pltpu.sync_copy(idx_h[b].at[pl.ds(tbase, TW)], idx_v[b])
    pltpu.sync_copy(ii_h.at[pl.ds(ebase, EW)], vi)
    pltpu.sync_copy(ij_h.at[pl.ds(ebase, EW)], vj)
    rows = [r0, r1, r2, r3, r4]

    def tri_fire(ci):
        toff = ci * CHT
        cps = []
        for b in range(5):
            tab = node_h if b < 3 else edge_h
            cps.append(pltpu.async_copy(
                tab.at[idx_v[b].at[pl.ds(toff, CHT)]], rows[b], sem))
        return cps, toff

    def tri_store(toff):
        for b in range(5):
            pltpu.async_copy(rows[b], x3_h.at[b, pl.ds(tbase + toff, CHT)],
                             semt)

    def drain_tri():
        for b in range(5):
            pltpu.make_async_copy(rows[b], x3_h.at[b, pl.ds(tbase, CHT)],
                                  semt).wait()

    def e_fire(ci):
        eoff = ci * CHE
        return [pltpu.async_copy(node_h.at[vi.at[pl.ds(eoff, CHE)]], e0,
                                 sem),
                pltpu.async_copy(node_h.at[vj.at[pl.ds(eoff, CHE)]], e1,
                                 sem)], eoff

    def e_store(eoff):
        pltpu.async_copy(e0, gi_h.at[pl.ds(ebase + eoff, CHE)], seme)
        pltpu.async_copy(e1, gj_h.at[pl.ds(ebase + eoff, CHE)], seme)

    def drain_e():
        pltpu.make_async_copy(e0, gi_h.at[pl.ds(ebase, CHE)], seme).wait()
        pltpu.make_async_copy(e1, gj_h.at[pl.ds(ebase, CHE)], seme).wait()

    def do_chunk(ci):
        cps, toff = tri_fire(ci)
        cpe, eoff = e_fire(ci)
        for cp in cps + cpe:
            cp.wait()
        tri_store(toff)
        e_store(eoff)

    def do_chunk_t(ci):
        cps, toff = tri_fire(ci)
        for cp in cps:
            cp.wait()
        tri_store(toff)

    do_chunk(0)

    def chunk(ci, carry):
        drain_tri()
        drain_e()
        do_chunk(ci)
        return carry
    lax.fori_loop(1, NCH_E, chunk, 0)
    drain_tri()
    drain_e()
    do_chunk_t(NCH_E)

    def chunk_t(ci, carry):
        drain_tri()
        do_chunk_t(ci)
        return carry
    lax.fori_loop(NCH_E + 1, NCH_T, chunk_t, 0)
    drain_tri()


def _gather_sc(node_embedding, edge_embedding, idxs, iis, ijs):
    f32 = jnp.float32
    return pl.kernel(
        _gather_body,
        out_type=(
            jax.ShapeDtypeStruct((5, TP, D), f32),
            jax.ShapeDtypeStruct((E, D), f32),
            jax.ShapeDtypeStruct((E, D), f32),
        ),
        mesh=_MESH,
        scratch_types=[
            pltpu.VMEM((TW,), jnp.int32),
            pltpu.VMEM((TW,), jnp.int32),
            pltpu.VMEM((TW,), jnp.int32),
            pltpu.VMEM((TW,), jnp.int32),
            pltpu.VMEM((TW,), jnp.int32),
            pltpu.VMEM((EW,), jnp.int32),
            pltpu.VMEM((EW,), jnp.int32),
            pltpu.VMEM((CHT, D), f32),
            pltpu.VMEM((CHT, D), f32),
            pltpu.VMEM((CHT, D), f32),
            pltpu.VMEM((CHT, D), f32),
            pltpu.VMEM((CHT, D), f32),
            pltpu.VMEM((CHE, D), f32),
            pltpu.VMEM((CHE, D), f32),
            pltpu.SemaphoreType.DMA,
            pltpu.SemaphoreType.DMA,
            pltpu.SemaphoreType.DMA,
        ],
    )(node_embedding, edge_embedding, *idxs, iis, ijs)


BT = 512


def _ln(x, g, b, eps=1e-5):
    mu = jnp.mean(x, axis=-1, keepdims=True)
    var = jnp.mean((x - mu) ** 2, axis=-1, keepdims=True)
    return (x - mu) * lax.rsqrt(var + eps) * g + b


def _c3_body(x_ref, w_ref, b_ref, g_ref, beta_ref, o_ref):
    acc = jnp.zeros((2 * D, BT), jnp.float32) + b_ref[...]
    for b in range(5):
        acc += lax.dot_general(w_ref[b], x_ref[b].astype(jnp.bfloat16),
                               (((1,), (1,)), ((), ())),
                               preferred_element_type=jnp.float32)
    mu = jnp.mean(acc, axis=0, keepdims=True)
    var = jnp.mean((acc - mu) ** 2, axis=0, keepdims=True)
    y = (acc - mu) * lax.rsqrt(var + 1e-5) * g_ref[...] + beta_ref[...]
    o_ref[...] = jax.nn.sigmoid(y[:D]) * jnp.tanh(y[D:])


def _c3_mlp_tc(x3, w3c, b3, g3, beta3):
    return pl.pallas_call(
        _c3_body,
        grid=(TP // BT,),
        in_specs=[
            pl.BlockSpec((5, BT, D), lambda i: (0, i, 0)),
            pl.BlockSpec((5, 2 * D, D), lambda i: (0, 0, 0)),
            pl.BlockSpec((2 * D, 1), lambda i: (0, 0)),
            pl.BlockSpec((2 * D, 1), lambda i: (0, 0)),
            pl.BlockSpec((2 * D, 1), lambda i: (0, 0)),
        ],
        out_specs=pl.BlockSpec((D, BT), lambda i: (0, i)),
        out_shape=jax.ShapeDtypeStruct((D, TP), jnp.float32),
    )(x3, w3c, b3, g3, beta3)


NCHU = TP // 128
CPW = NCHU // NS
NPASS = D // (8 * NC)
EWR = E // NS


def _scatter_body(gt_h, idx2_h, zero_h, s1_h,
                  idx2_v, gva, gvb, zb, wb, dr,
                  a0, a1, a2, a3, a4, a5, a6, a7, seml, sema, semb):
    c = lax.axis_index("c")
    s = lax.axis_index("s")
    accs = [a0, a1, a2, a3, a4, a5, a6, a7]
    gv = [gva, gvb]
    sems = [sema, semb]
    pltpu.sync_copy(idx2_h.at[pl.ds(s * CPW, CPW)], idx2_v)
    pltpu.sync_copy(zero_h, zb)

    for p in range(NPASS):
        k = p * NC + c
        f0 = pl.multiple_of(k * 8, 8)
        for f in range(8):
            pltpu.sync_copy(zb.at[pl.ds(0, EWR)],
                            accs[f].at[pl.ds(s * EWR, EWR)])
        plsc.subcore_barrier()

        def load(j, buf):
            ch = s * CPW + j
            return pltpu.async_copy(
                gt_h.at[pl.ds(f0, 8), pl.ds(ch * 128, 128)], buf, seml)

        def fire_adds(j, b):
            for f in range(8):
                pltpu.async_copy(gv[b].at[f], accs[f].at[idx2_v.at[j]],
                                 sems[b], add=True)

        def drain_adds(b):
            for f in range(8):
                pltpu.make_async_copy(zero_h.at[pl.ds(0, 128)], dr,
                                      sems[b]).wait()

        for b in range(2):
            load(b, gv[b]).wait()
            fire_adds(b, b)

        def pair(g, carry):
            for b in range(2):
                j = g * 2 + b
                drain_adds(b)
                load(j, gv[b]).wait()
                fire_adds(j, b)
            return carry
        lax.fori_loop(1, CPW // 2, pair, 0)
        for b in range(2):
            drain_adds(b)
        plsc.subcore_barrier()

        for f in range(8):
            pltpu.sync_copy(accs[f].at[pl.ds(s * EWR, EWR)], wb)
            pltpu.sync_copy(wb, s1_h.at[pl.ds((k * 8 + f) * E + s * EWR,
                                              EWR)])
        plsc.subcore_barrier()


def _scatter_sc(gt, idx2):
    f32 = jnp.float32
    zero = jnp.zeros((EWR,), f32)
    acc_t = pltpu.VMEM_SHARED((E + 128,), f32)
    return pl.kernel(
        _scatter_body,
        out_type=jax.ShapeDtypeStruct((D * E,), f32),
        mesh=_MESH,
        scratch_types=[
            pltpu.VMEM((CPW, 128), jnp.int32),
            pltpu.VMEM((8, 128), f32),
            pltpu.VMEM((8, 128), f32),
            pltpu.VMEM((EWR,), f32),
            pltpu.VMEM((EWR,), f32),
            pltpu.VMEM((128,), f32),
            acc_t, acc_t, acc_t, acc_t, acc_t, acc_t, acc_t, acc_t,
            pltpu.SemaphoreType.DMA,
            pltpu.SemaphoreType.DMA,
            pltpu.SemaphoreType.DMA,
        ],
    )(gt, idx2, zero)


BE = 640


def _final_body(gi_ref, gj_ref, s_ref, edge_ref, w2_ref, b2_ref, g2_ref,
                beta2_ref, g22_ref, beta22_ref, g32_ref, beta32_ref, o_ref):
    prod = gi_ref[...] * gj_ref[...]
    y = lax.dot_general(prod, w2_ref[...], (((1,), (0,)), ((), ())),
                        preferred_element_type=jnp.float32) + b2_ref[...]
    y = _ln(y, g2_ref[...], beta2_ref[...])
    c2e = _ln(jax.nn.sigmoid(y[:, :D]) * jnp.tanh(y[:, D:]),
              g22_ref[...], beta22_ref[...])
    c3e = _ln(s_ref[...], g32_ref[...], beta32_ref[...])
    o_ref[...] = jnp.tanh(edge_ref[...] + c2e + c3e)


def _final_tc(gi, gj, ssum, edge, w2r, b2, g2, beta2, g22, beta22, g32,
              beta32):
    vspec = lambda n: pl.BlockSpec((n,), lambda i: (0,))
    bspec = pl.BlockSpec((BE, D), lambda i: (i, 0))
    return pl.pallas_call(
        _final_body,
        grid=(E // BE,),
        in_specs=[
            bspec, bspec, bspec, bspec,
            pl.BlockSpec((D, 2 * D), lambda i: (0, 0)),
            vspec(2 * D), vspec(2 * D), vspec(2 * D),
            vspec(D), vspec(D), vspec(D), vspec(D),
        ],
        out_specs=bspec,
        out_shape=jax.ShapeDtypeStruct((E, D), jnp.float32),
    )(gi, gj, ssum, edge, w2r, b2, g2, beta2, g22, beta22, g32, beta32)


@jax.jit
def kernel(node_embedding, edge_embedding, i, j, index_i, index_j, index_k,
           index_ji, index_kj, lin_c2_w, lin_c2_b, lin_c3_w, lin_c3_b,
           ln_c2_g, ln_c2_b, ln_c3_g, ln_c3_b,
           ln_c2_2_g, ln_c2_2_b, ln_c3_2_g, ln_c3_2_b):
    pad = jnp.zeros((TP - T,), jnp.int32)
    idxs = [jnp.concatenate([x.astype(jnp.int32), pad])
            for x in (index_i, index_j, index_k, index_ji, index_kj)]
    x3, gi, gj = _gather_sc(node_embedding, edge_embedding, idxs,
                            i.astype(jnp.int32), j.astype(jnp.int32))

    w3c = lin_c3_w.reshape(2 * D, 5, D).transpose(1, 0, 2).astype(
        jnp.bfloat16)
    gt = _c3_mlp_tc(x3, w3c, lin_c3_b.reshape(2 * D, 1),
                    ln_c3_g.reshape(2 * D, 1), ln_c3_b.reshape(2 * D, 1))

    idx2 = jnp.concatenate(
        [index_ji.astype(jnp.int32),
         jnp.full((TP - T,), E, jnp.int32)]).reshape(NCHU, 128)
    s1 = _scatter_sc(gt, idx2)
    ssum = s1.reshape(D, E).T

    return _final_tc(gi, gj, ssum, edge_embedding, lin_c2_w.T, lin_c2_b,
                     ln_c2_g, ln_c2_b, ln_c2_2_g, ln_c2_2_b,
                     ln_c3_2_g, ln_c3_2_b)


# speedup vs baseline: 2.0896x; 1.1104x over previous
import jax
import jax.numpy as jnp
from jax import lax
from jax.experimental import pallas as pl
from jax.experimental.pallas import tpu as pltpu
from jax.experimental.pallas import tpu_sc as plsc

N = 10000
E = 160000
T = 320000
D = 128

NC = 2
NS = 16
NW = NC * NS

TP = 327680
TH = TP // 2
TWH = TH // NW
EW = E // NW
CHT = 80
CHE = 40
NCH_E = EW // CHE
NCH_T = TWH // CHT

_MESH = plsc.VectorSubcoreMesh(core_axis_name="c", subcore_axis_name="s")


def _make_gather_body(half):
    with_edges = half == 0

    def body(node_h, edge_h, i0_h, i1_h, i2_h, i3_h, i4_h, ii_h, ij_h,
             *out_and_scratch):
        if with_edges:
            (x3_h, gi_h, gj_h, v0, v1, v2, v3, v4, vi, vj,
             r0, r1, r2, r3, r4, e0, e1, sem, semt, seme) = out_and_scratch
        else:
            (x3_h, v0, v1, v2, v3, v4, vi, vj,
             r0, r1, r2, r3, r4, e0, e1, sem, semt, seme) = out_and_scratch
        c = lax.axis_index("c")
        s = lax.axis_index("s")
        wid = s * NC + c
        tbase = wid * TWH
        ebase = wid * EW
        idx_h = [i0_h, i1_h, i2_h, i3_h, i4_h]
        idx_v = [v0, v1, v2, v3, v4]
        for b in range(5):
            pltpu.sync_copy(idx_h[b].at[pl.ds(half * TH + tbase, TWH)],
                            idx_v[b])
        if with_edges:
            pltpu.sync_copy(ii_h.at[pl.ds(ebase, EW)], vi)
            pltpu.sync_copy(ij_h.at[pl.ds(ebase, EW)], vj)
        rows = [r0, r1, r2, r3, r4]

        def tri_fire(ci):
            toff = ci * CHT
            cps = []
            for b in range(5):
                tab = node_h if b < 3 else edge_h
                cps.append(pltpu.async_copy(
                    tab.at[idx_v[b].at[pl.ds(toff, CHT)]], rows[b], sem))
            return cps, toff

        def tri_store(toff):
            for b in range(5):
                pltpu.async_copy(rows[b],
                                 x3_h.at[b, pl.ds(tbase + toff, CHT)], semt)

        def drain_tri():
            for b in range(5):
                pltpu.make_async_copy(
                    rows[b], x3_h.at[b, pl.ds(tbase, CHT)], semt).wait()

        if with_edges:
            def e_fire(ci):
                eoff = ci * CHE
                return [pltpu.async_copy(
                            node_h.at[vi.at[pl.ds(eoff, CHE)]], e0, sem),
                        pltpu.async_copy(
                            node_h.at[vj.at[pl.ds(eoff, CHE)]], e1, sem)
                        ], eoff

            def e_store(eoff):
                pltpu.async_copy(e0, gi_h.at[pl.ds(ebase + eoff, CHE)],
                                 seme)
                pltpu.async_copy(e1, gj_h.at[pl.ds(ebase + eoff, CHE)],
                                 seme)

            def drain_e():
                pltpu.make_async_copy(e0, gi_h.at[pl.ds(ebase, CHE)],
                                      seme).wait()
                pltpu.make_async_copy(e1, gj_h.at[pl.ds(ebase, CHE)],
                                      seme).wait()

            def do_chunk(ci):
                cps, toff = tri_fire(ci)
                cpe, eoff = e_fire(ci)
                for cp in cps + cpe:
                    cp.wait()
                tri_store(toff)
                e_store(eoff)

            def do_echunk(ci):
                cpe, eoff = e_fire(ci)
                for cp in cpe:
                    cp.wait()
                e_store(eoff)

            do_chunk(0)

            def chunk(ci, carry):
                drain_tri()
                drain_e()
                do_chunk(ci)
                return carry
            lax.fori_loop(1, NCH_T, chunk, 0)
            drain_tri()
            drain_e()
            do_echunk(NCH_T)

            def echunk(ci, carry):
                drain_e()
                do_echunk(ci)
                return carry
            lax.fori_loop(NCH_T + 1, NCH_E, echunk, 0)
            drain_e()
        else:
            def do_chunk_t(ci):
                cps, toff = tri_fire(ci)
                for cp in cps:
                    cp.wait()
                tri_store(toff)

            do_chunk_t(0)

            def chunk_t(ci, carry):
                drain_tri()
                do_chunk_t(ci)
                return carry
            lax.fori_loop(1, NCH_T, chunk_t, 0)
            drain_tri()

    return body


def _gather_sc(half, node_embedding, edge_embedding, idxs, iis, ijs):
    f32 = jnp.float32
    out_type = [jax.ShapeDtypeStruct((5, TH, D), f32)]
    if half == 0:
        out_type += [jax.ShapeDtypeStruct((E, D), f32),
                     jax.ShapeDtypeStruct((E, D), f32)]
    return pl.kernel(
        _make_gather_body(half),
        out_type=tuple(out_type),
        mesh=_MESH,
        scratch_types=[
            pltpu.VMEM((TWH,), jnp.int32),
            pltpu.VMEM((TWH,), jnp.int32),
            pltpu.VMEM((TWH,), jnp.int32),
            pltpu.VMEM((TWH,), jnp.int32),
            pltpu.VMEM((TWH,), jnp.int32),
            pltpu.VMEM((EW,), jnp.int32),
            pltpu.VMEM((EW,), jnp.int32),
            pltpu.VMEM((CHT, D), f32),
            pltpu.VMEM((CHT, D), f32),
            pltpu.VMEM((CHT, D), f32),
            pltpu.VMEM((CHT, D), f32),
            pltpu.VMEM((CHT, D), f32),
            pltpu.VMEM((CHE, D), f32),
            pltpu.VMEM((CHE, D), f32),
            pltpu.SemaphoreType.DMA,
            pltpu.SemaphoreType.DMA,
            pltpu.SemaphoreType.DMA,
        ],
    )(node_embedding, edge_embedding, *idxs, iis, ijs)


BT = 512


def _ln(x, g, b, eps=1e-5):
    mu = jnp.mean(x, axis=-1, keepdims=True)
    var = jnp.mean((x - mu) ** 2, axis=-1, keepdims=True)
    return (x - mu) * lax.rsqrt(var + eps) * g + b


def _c3_body(x_ref, w_ref, b_ref, g_ref, beta_ref, o_ref):
    acc = jnp.zeros((2 * D, BT), jnp.float32) + b_ref[...]
    for b in range(5):
        acc += lax.dot_general(w_ref[b], x_ref[b].astype(jnp.bfloat16),
                               (((1,), (1,)), ((), ())),
                               preferred_element_type=jnp.float32)
    mu = jnp.mean(acc, axis=0, keepdims=True)
    var = jnp.mean((acc - mu) ** 2, axis=0, keepdims=True)
    y = (acc - mu) * lax.rsqrt(var + 1e-5) * g_ref[...] + beta_ref[...]
    o_ref[...] = jax.nn.sigmoid(y[:D]) * jnp.tanh(y[D:])


def _c3_mlp_tc(x3, w3c, b3, g3, beta3):
    return pl.pallas_call(
        _c3_body,
        grid=(TH // BT,),
        in_specs=[
            pl.BlockSpec((5, BT, D), lambda i: (0, i, 0)),
            pl.BlockSpec((5, 2 * D, D), lambda i: (0, 0, 0)),
            pl.BlockSpec((2 * D, 1), lambda i: (0, 0)),
            pl.BlockSpec((2 * D, 1), lambda i: (0, 0)),
            pl.BlockSpec((2 * D, 1), lambda i: (0, 0)),
        ],
        out_specs=pl.BlockSpec((D, BT), lambda i: (0, i)),
        out_shape=jax.ShapeDtypeStruct((D, TH), jnp.float32),
    )(x3, w3c, b3, g3, beta3)


NCHU = TP // 128
CPW = NCHU // NS
NPASS = D // (8 * NC)
EWR = E // NS


def _scatter_body(gta_h, gtb_h, idx2_h, zero_h, s1_h,
                  idx2_v, gva, gvb, zb, wb, dr,
                  a0, a1, a2, a3, a4, a5, a6, a7, seml, sema, semb):
    c = lax.axis_index("c")
    s = lax.axis_index("s")
    accs = [a0, a1, a2, a3, a4, a5, a6, a7]
    gv = [gva, gvb]
    sems = [sema, semb]
    pltpu.sync_copy(idx2_h.at[pl.ds(s * CPW, CPW)], idx2_v)
    pltpu.sync_copy(zero_h, zb)
    in_a = s < (NS // 2)

    for p in range(NPASS):
        k = p * NC + c
        f0 = pl.multiple_of(k * 8, 8)
        for f in range(8):
            pltpu.sync_copy(zb.at[pl.ds(0, EWR)],
                            accs[f].at[pl.ds(s * EWR, EWR)])
        plsc.subcore_barrier()

        def load(j, buf):
            ch = s * CPW + j

            @pl.when(in_a)
            def _():
                pltpu.async_copy(
                    gta_h.at[pl.ds(f0, 8), pl.ds(ch * 128, 128)], buf,
                    seml)

            @pl.when(jnp.logical_not(in_a))
            def _():
                pltpu.async_copy(
                    gtb_h.at[pl.ds(f0, 8), pl.ds(ch * 128 - TH, 128)],
                    buf, seml)

        def wait_load(buf):
            pltpu.make_async_copy(
                gta_h.at[pl.ds(0, 8), pl.ds(0, 128)], buf, seml).wait()

        def fire_adds(j, b):
            for f in range(8):
                pltpu.async_copy(gv[b].at[f], accs[f].at[idx2_v.at[j]],
                                 sems[b], add=True)

        def drain_adds(b):
            for f in range(8):
                pltpu.make_async_copy(zero_h.at[pl.ds(0, 128)], dr,
                                      sems[b]).wait()

        for b in range(2):
            load(b, gv[b])
            wait_load(gv[b])
            fire_adds(b, b)

        def pair(g, carry):
            for b in range(2):
                j = g * 2 + b
                drain_adds(b)
                load(j, gv[b])
                wait_load(gv[b])
                fire_adds(j, b)
            return carry
        lax.fori_loop(1, CPW // 2, pair, 0)
        for b in range(2):
            drain_adds(b)
        plsc.subcore_barrier()

        for f in range(8):
            pltpu.sync_copy(accs[f].at[pl.ds(s * EWR, EWR)], wb)
            pltpu.sync_copy(wb, s1_h.at[pl.ds((k * 8 + f) * E + s * EWR,
                                              EWR)])
        plsc.subcore_barrier()


def _scatter_sc(gta, gtb, idx2):
    f32 = jnp.float32
    zero = jnp.zeros((EWR,), f32)
    acc_t = pltpu.VMEM_SHARED((E + 128,), f32)
    return pl.kernel(
        _scatter_body,
        out_type=jax.ShapeDtypeStruct((D * E,), f32),
        mesh=_MESH,
        scratch_types=[
            pltpu.VMEM((CPW, 128), jnp.int32),
            pltpu.VMEM((8, 128), f32),
            pltpu.VMEM((8, 128), f32),
            pltpu.VMEM((EWR,), f32),
            pltpu.VMEM((EWR,), f32),
            pltpu.VMEM((128,), f32),
            acc_t, acc_t, acc_t, acc_t, acc_t, acc_t, acc_t, acc_t,
            pltpu.SemaphoreType.DMA,
            pltpu.SemaphoreType.DMA,
            pltpu.SemaphoreType.DMA,
        ],
    )(gta, gtb, idx2, zero)


BE = 640


def _final_body(gi_ref, gj_ref, s_ref, edge_ref, w2_ref, b2_ref, g2_ref,
                beta2_ref, g22_ref, beta22_ref, g32_ref, beta32_ref, o_ref):
    prod = gi_ref[...] * gj_ref[...]
    y = lax.dot_general(prod, w2_ref[...], (((1,), (0,)), ((), ())),
                        preferred_element_type=jnp.float32) + b2_ref[...]
    y = _ln(y, g2_ref[...], beta2_ref[...])
    c2e = _ln(jax.nn.sigmoid(y[:, :D]) * jnp.tanh(y[:, D:]),
              g22_ref[...], beta22_ref[...])
    c3e = _ln(s_ref[...], g32_ref[...], beta32_ref[...])
    o_ref[...] = jnp.tanh(edge_ref[...] + c2e + c3e)


def _final_tc(gi, gj, ssum, edge, w2r, b2, g2, beta2, g22, beta22, g32,
              beta32):
    vspec = lambda n: pl.BlockSpec((n,), lambda i: (0,))
    bspec = pl.BlockSpec((BE, D), lambda i: (i, 0))
    return pl.pallas_call(
        _final_body,
        grid=(E // BE,),
        in_specs=[
            bspec, bspec, bspec, bspec,
            pl.BlockSpec((D, 2 * D), lambda i: (0, 0)),
            vspec(2 * D), vspec(2 * D), vspec(2 * D),
            vspec(D), vspec(D), vspec(D), vspec(D),
        ],
        out_specs=bspec,
        out_shape=jax.ShapeDtypeStruct((E, D), jnp.float32),
    )(gi, gj, ssum, edge, w2r, b2, g2, beta2, g22, beta22, g32, beta32)


@jax.jit
def kernel(node_embedding, edge_embedding, i, j, index_i, index_j, index_k,
           index_ji, index_kj, lin_c2_w, lin_c2_b, lin_c3_w, lin_c3_b,
           ln_c2_g, ln_c2_b, ln_c3_g, ln_c3_b,
           ln_c2_2_g, ln_c2_2_b, ln_c3_2_g, ln_c3_2_b):
    pad = jnp.zeros((TP - T,), jnp.int32)
    idxs = [jnp.concatenate([x.astype(jnp.int32), pad])
            for x in (index_i, index_j, index_k, index_ji, index_kj)]
    ii = i.astype(jnp.int32)
    ij = j.astype(jnp.int32)
    x3a, gi, gj = _gather_sc(0, node_embedding, edge_embedding, idxs,
                             ii, ij)
    x3b, = _gather_sc(1, node_embedding, edge_embedding, idxs, ii, ij)

    w3c = lin_c3_w.reshape(2 * D, 5, D).transpose(1, 0, 2).astype(
        jnp.bfloat16)
    b3 = lin_c3_b.reshape(2 * D, 1)
    g3 = ln_c3_g.reshape(2 * D, 1)
    be3 = ln_c3_b.reshape(2 * D, 1)
    gta = _c3_mlp_tc(x3a, w3c, b3, g3, be3)
    gtb = _c3_mlp_tc(x3b, w3c, b3, g3, be3)

    idx2 = jnp.concatenate(
        [index_ji.astype(jnp.int32),
         jnp.full((TP - T,), E, jnp.int32)]).reshape(NCHU, 128)
    s1 = _scatter_sc(gta, gtb, idx2)
    ssum = s1.reshape(D, E).T

    return _final_tc(gi, gj, ssum, edge_embedding, lin_c2_w.T, lin_c2_b,
                     ln_c2_g, ln_c2_b, ln_c2_2_g, ln_c2_2_b,
                     ln_c3_2_g, ln_c3_2_b)
